# baseline (speedup 1.0000x reference)
import functools

import jax
import jax.numpy as jnp
from jax import lax
from jax.experimental import pallas as pl
from jax.experimental.pallas import tpu as pltpu
from jax.experimental.pallas import tpu_sc as plsc

_B = 16384
_A = 128
_NC = 2
_NS = 16
_NW = _NC * _NS
_BPW = _B // _NW
_CHUNK = 128
_NCHUNK = _BPW // _CHUNK
_L = 16

_mesh = plsc.VectorSubcoreMesh(core_axis_name="c", subcore_axis_name="s")


@functools.partial(
    pl.kernel,
    out_type=jax.ShapeDtypeStruct((_B,), jnp.int32),
    mesh=_mesh,
    compiler_params=pltpu.CompilerParams(needs_layout_passes=False),
    scratch_types=[
        pltpu.VMEM((_BPW,), jnp.int32),
        pltpu.VMEM((_CHUNK, _A + 1), jnp.float32),
        pltpu.VMEM((_CHUNK, _A + 1), jnp.float32),
        pltpu.VMEM((_BPW,), jnp.int32),
        pltpu.SemaphoreType.DMA,
        pltpu.SemaphoreType.DMA,
    ],
)
def _argmax_gather(states_hbm, policy_hbm, out_hbm,
                   idx_v, buf0, buf1, out_v, sem0, sem1):
    wid = lax.axis_index("s") * _NC + lax.axis_index("c")
    base = wid * _BPW
    pltpu.sync_copy(states_hbm.at[pl.ds(base, _BPW)], idx_v)

    bufs = (buf0, buf1)
    sems = (sem0, sem1)

    def start(k):
        return pltpu.async_copy(
            policy_hbm.at[idx_v.at[pl.ds(k * _CHUNK, _CHUNK)]],
            bufs[k % 2].at[:, pl.ds(0, _A)], sems[k % 2])

    def compute(k):
        buf = bufs[k % 2]

        def group_body(g, _):
            row_ids = lax.iota(jnp.int32, _L) + g * _L
            bv = plsc.load_gather(buf, [row_ids, jnp.zeros((_L,), jnp.int32)])
            bi = jnp.zeros((_L,), jnp.int32)
            for c in range(1, _A):
                v = plsc.load_gather(buf, [row_ids, jnp.full((_L,), c, jnp.int32)])
                gt = v > bv
                bv = jnp.where(gt, v, bv)
                bi = jnp.where(gt, c, bi)
            out_v[pl.ds(k * _CHUNK + g * _L, _L)] = bi
            return 0

        lax.fori_loop(0, _CHUNK // _L, group_body, 0)

    cp = start(0)
    for k in range(_NCHUNK):
        nxt = start(k + 1) if k + 1 < _NCHUNK else None
        cp.wait()
        compute(k)
        cp = nxt

    pltpu.sync_copy(out_v, out_hbm.at[pl.ds(base, _BPW)])


def kernel(states, policy):
    return _argmax_gather(states.astype(jnp.int32), policy)


# speedup vs baseline: 2.0412x; 2.0412x over previous
import functools

import jax
import jax.numpy as jnp
from jax import lax
from jax.experimental import pallas as pl
from jax.experimental.pallas import tpu as pltpu
from jax.experimental.pallas import tpu_sc as plsc

_B = 16384
_A = 128
_NC = 2
_NS = 16
_NW = _NC * _NS
_BPW = _B // _NW
_CHUNK = 128
_NCHUNK = _BPW // _CHUNK
_L = 16

_mesh = plsc.VectorSubcoreMesh(core_axis_name="c", subcore_axis_name="s")


@functools.partial(
    pl.kernel,
    out_type=jax.ShapeDtypeStruct((_B,), jnp.int32),
    mesh=_mesh,
    compiler_params=pltpu.CompilerParams(needs_layout_passes=False),
    scratch_types=[
        pltpu.VMEM((_BPW,), jnp.int32),
        pltpu.VMEM((_CHUNK, _A), jnp.float32),
        pltpu.VMEM((_CHUNK, _A), jnp.float32),
        pltpu.VMEM((_BPW,), jnp.int32),
        pltpu.SemaphoreType.DMA,
        pltpu.SemaphoreType.DMA,
    ],
)
def _argmax_gather(states_hbm, policy_hbm, out_hbm,
                   idx_v, buf0, buf1, out_v, sem0, sem1):
    wid = lax.axis_index("s") * _NC + lax.axis_index("c")
    base = wid * _BPW
    pltpu.sync_copy(states_hbm.at[pl.ds(base, _BPW)], idx_v)

    bufs = (buf0, buf1)
    sems = (sem0, sem1)

    def start(k):
        return pltpu.async_copy(
            policy_hbm.at[idx_v.at[pl.ds(k * _CHUNK, _CHUNK)]],
            bufs[k % 2], sems[k % 2])

    def compute(k):
        buf = bufs[k % 2]

        def group_body(g, _):
            row_ids = lax.iota(jnp.int32, _L) + g * _L
            col = lax.iota(jnp.int32, _L)
            bv = plsc.load_gather(buf, [row_ids, col])
            bi = col
            for _ in range(1, _A):
                col = (col + 1) & (_A - 1)
                v = plsc.load_gather(buf, [row_ids, col])
                upd = (v > bv) | ((v == bv) & (col < bi))
                bv = jnp.where(upd, v, bv)
                bi = jnp.where(upd, col, bi)
            out_v[pl.ds(k * _CHUNK + g * _L, _L)] = bi
            return 0

        lax.fori_loop(0, _CHUNK // _L, group_body, 0)

    cp = start(0)
    for k in range(_NCHUNK):
        nxt = start(k + 1) if k + 1 < _NCHUNK else None
        cp.wait()
        compute(k)
        cp = nxt

    pltpu.sync_copy(out_v, out_hbm.at[pl.ds(base, _BPW)])


def kernel(states, policy):
    return _argmax_gather(states.astype(jnp.int32), policy)


# speedup vs baseline: 2.4191x; 1.1852x over previous
import functools

import jax
import jax.numpy as jnp
from jax import lax
from jax.experimental import pallas as pl
from jax.experimental.pallas import tpu as pltpu
from jax.experimental.pallas import tpu_sc as plsc

_B = 16384
_A = 128
_NC = 2
_NS = 16
_NW = _NC * _NS
_BPW = _B // _NW
_CHUNK = 128
_NCHUNK = _BPW // _CHUNK
_L = 16
_NCHAIN = 4

_mesh = plsc.VectorSubcoreMesh(core_axis_name="c", subcore_axis_name="s")


@functools.partial(
    pl.kernel,
    out_type=jax.ShapeDtypeStruct((_B,), jnp.int32),
    mesh=_mesh,
    compiler_params=pltpu.CompilerParams(needs_layout_passes=False),
    scratch_types=[
        pltpu.VMEM((_BPW,), jnp.int32),
        pltpu.VMEM((_CHUNK, _A), jnp.float32),
        pltpu.VMEM((_CHUNK, _A), jnp.float32),
        pltpu.VMEM((_BPW,), jnp.int32),
        pltpu.SemaphoreType.DMA,
        pltpu.SemaphoreType.DMA,
    ],
)
def _argmax_gather(states_hbm, policy_hbm, out_hbm,
                   idx_v, buf0, buf1, out_v, sem0, sem1):
    wid = lax.axis_index("s") * _NC + lax.axis_index("c")
    base = wid * _BPW
    pltpu.sync_copy(states_hbm.at[pl.ds(base, _BPW)], idx_v)

    bufs = (buf0, buf1)
    sems = (sem0, sem1)

    def start(k):
        return pltpu.async_copy(
            policy_hbm.at[idx_v.at[pl.ds(k * _CHUNK, _CHUNK)]],
            bufs[k % 2], sems[k % 2])

    def compute(k):
        buf = bufs[k % 2]

        def group_body(g, _):
            row_ids = lax.iota(jnp.int32, _L) + g * _L
            cols = [None] * _NCHAIN
            bvs = [None] * _NCHAIN
            bis = [None] * _NCHAIN
            for j in range(_NCHAIN):
                cols[j] = lax.iota(jnp.int32, _L) + j * (_A // _NCHAIN)
                bvs[j] = plsc.load_gather(buf, [row_ids, cols[j]])
                bis[j] = cols[j]
            for _ in range(1, _A // _NCHAIN):
                for j in range(_NCHAIN):
                    cols[j] = (cols[j] + 1) & (_A - 1)
                    v = plsc.load_gather(buf, [row_ids, cols[j]])
                    upd = (v > bvs[j]) | ((v == bvs[j]) & (cols[j] < bis[j]))
                    bvs[j] = jnp.where(upd, v, bvs[j])
                    bis[j] = jnp.where(upd, cols[j], bis[j])
            step = 1
            while step < _NCHAIN:
                for j in range(0, _NCHAIN, 2 * step):
                    v, c = bvs[j + step], bis[j + step]
                    upd = (v > bvs[j]) | ((v == bvs[j]) & (c < bis[j]))
                    bvs[j] = jnp.where(upd, v, bvs[j])
                    bis[j] = jnp.where(upd, c, bis[j])
                step *= 2
            out_v[pl.ds(k * _CHUNK + g * _L, _L)] = bis[0]
            return 0

        lax.fori_loop(0, _CHUNK // _L, group_body, 0)

    cp = start(0)
    for k in range(_NCHUNK):
        nxt = start(k + 1) if k + 1 < _NCHUNK else None
        cp.wait()
        compute(k)
        cp = nxt

    pltpu.sync_copy(out_v, out_hbm.at[pl.ds(base, _BPW)])


def kernel(states, policy):
    return _argmax_gather(states.astype(jnp.int32), policy)


# speedup vs baseline: 2.4786x; 1.0246x over previous
import functools

import jax
import jax.numpy as jnp
from jax import lax
from jax.experimental import pallas as pl
from jax.experimental.pallas import tpu as pltpu
from jax.experimental.pallas import tpu_sc as plsc

_B = 16384
_A = 128
_NC = 2
_NS = 16
_NW = _NC * _NS
_BPW = _B // _NW
_CHUNK = 128
_NCHUNK = _BPW // _CHUNK
_L = 16
_NCHAIN = 8

_mesh = plsc.VectorSubcoreMesh(core_axis_name="c", subcore_axis_name="s")


@functools.partial(
    pl.kernel,
    out_type=jax.ShapeDtypeStruct((_B,), jnp.int32),
    mesh=_mesh,
    compiler_params=pltpu.CompilerParams(needs_layout_passes=False),
    scratch_types=[
        pltpu.VMEM((_BPW,), jnp.int32),
        pltpu.VMEM((_CHUNK, _A), jnp.float32),
        pltpu.VMEM((_CHUNK, _A), jnp.float32),
        pltpu.VMEM((_BPW,), jnp.int32),
        pltpu.SemaphoreType.DMA,
        pltpu.SemaphoreType.DMA,
    ],
)
def _argmax_gather(states_hbm, policy_hbm, out_hbm,
                   idx_v, buf0, buf1, out_v, sem0, sem1):
    wid = lax.axis_index("s") * _NC + lax.axis_index("c")
    base = wid * _BPW
    pltpu.sync_copy(states_hbm.at[pl.ds(base, _BPW)], idx_v)

    bufs = (buf0, buf1)
    sems = (sem0, sem1)

    def start(k):
        return pltpu.async_copy(
            policy_hbm.at[idx_v.at[pl.ds(k * _CHUNK, _CHUNK)]],
            bufs[k % 2], sems[k % 2])

    def compute(k):
        buf = bufs[k % 2]

        def group_body(g, _):
            row_ids = lax.iota(jnp.int32, _L) + g * _L
            cols = [None] * _NCHAIN
            bvs = [None] * _NCHAIN
            bis = [None] * _NCHAIN
            for j in range(_NCHAIN):
                cols[j] = lax.iota(jnp.int32, _L) + j * (_A // _NCHAIN)
                bvs[j] = plsc.load_gather(buf, [row_ids, cols[j]])
                bis[j] = cols[j]
            for _ in range(1, _A // _NCHAIN):
                for j in range(_NCHAIN):
                    cols[j] = (cols[j] + 1) & (_A - 1)
                    v = plsc.load_gather(buf, [row_ids, cols[j]])
                    upd = (v > bvs[j]) | ((v == bvs[j]) & (cols[j] < bis[j]))
                    bvs[j] = jnp.where(upd, v, bvs[j])
                    bis[j] = jnp.where(upd, cols[j], bis[j])
            step = 1
            while step < _NCHAIN:
                for j in range(0, _NCHAIN, 2 * step):
                    v, c = bvs[j + step], bis[j + step]
                    upd = (v > bvs[j]) | ((v == bvs[j]) & (c < bis[j]))
                    bvs[j] = jnp.where(upd, v, bvs[j])
                    bis[j] = jnp.where(upd, c, bis[j])
                step *= 2
            out_v[pl.ds(k * _CHUNK + g * _L, _L)] = bis[0]
            return 0

        lax.fori_loop(0, _CHUNK // _L, group_body, 0)

    cp = start(0)
    for k in range(_NCHUNK):
        nxt = start(k + 1) if k + 1 < _NCHUNK else None
        cp.wait()
        compute(k)
        cp = nxt

    pltpu.sync_copy(out_v, out_hbm.at[pl.ds(base, _BPW)])


def kernel(states, policy):
    return _argmax_gather(states.astype(jnp.int32), policy)
